# baseline (speedup 1.0000x reference)
import math

import jax
import jax.numpy as jnp
import numpy as np
from jax import lax
from jax.experimental import pallas as pl
from jax.experimental.pallas import tpu as pltpu
from jax.experimental.pallas import tpu_sc as plsc

_B = 4
_S = 4096
_D = 1024
_MAX_LEN = 4096
_NW = 32
_RPW = _S // _NW
_RB = 16
_NK = _RPW // _RB
_BLK = _RB * _D


def _sinusoid_table():
    position = np.arange(0, _MAX_LEN, dtype=np.float64)[:, None]
    div_term = np.exp(np.arange(0, _D, 2, dtype=np.float64)
                      * (-math.log(10000.0) / _D))
    enc = np.zeros((_MAX_LEN, _D), dtype=np.float32)
    enc[:, 0::2] = np.sin(position * div_term)
    enc[:, 1::2] = np.cos(position * div_term)
    return jnp.asarray(enc.reshape(-1))


def _sc_body(x_hbm, emb_hbm, enc_hbm, out_hbm, emb16_v, pe_v, x_v, o_v):
    c = lax.axis_index("c")
    s = lax.axis_index("s")
    wid = s * 2 + c
    base = wid * _RPW * _D

    for r in range(_RB):
        pltpu.sync_copy(emb_hbm.at[0], emb16_v.at[pl.ds(r * _D, _D)])

    for k in range(_NK):
        off = base + k * _BLK
        pltpu.sync_copy(enc_hbm.at[pl.ds(off, _BLK)], pe_v)

        @plsc.parallel_loop(0, _BLK, step=16, unroll=8)
        def _(i):
            pe_v[pl.ds(i, 16)] = pe_v[pl.ds(i, 16)] + emb16_v[pl.ds(i, 16)]

        for b in range(_B):
            pltpu.sync_copy(x_hbm.at[b, pl.ds(off, _BLK)], x_v)

            @plsc.parallel_loop(0, _BLK, step=16, unroll=8)
            def _(i):
                o_v[pl.ds(i, 16)] = x_v[pl.ds(i, 16)] + pe_v[pl.ds(i, 16)]

            pltpu.sync_copy(o_v, out_hbm.at[b, pl.ds(off, _BLK)])


def kernel(x, node_emb):
    enc = _sinusoid_table()
    x2 = x.reshape(_B, _S * _D)
    out = pl.kernel(
        _sc_body,
        out_type=jax.ShapeDtypeStruct((_B, _S * _D), jnp.float32),
        mesh=plsc.VectorSubcoreMesh(core_axis_name="c", subcore_axis_name="s"),
        scratch_types=[
            pltpu.VMEM((_BLK,), jnp.float32),
            pltpu.VMEM((_BLK,), jnp.float32),
            pltpu.VMEM((_BLK,), jnp.float32),
            pltpu.VMEM((_BLK,), jnp.float32),
        ],
    )(x2, node_emb, enc)
    return out.reshape(_B, _S, _D)


# speedup vs baseline: 1.0736x; 1.0736x over previous
import math

import jax
import jax.numpy as jnp
import numpy as np
from jax import lax
from jax.experimental import pallas as pl
from jax.experimental.pallas import tpu as pltpu
from jax.experimental.pallas import tpu_sc as plsc

_B = 4
_S = 4096
_D = 1024
_MAX_LEN = 4096
_NW = 32
_RPW = _S // _NW
_RB = 16
_NK = _RPW // _RB
_BLK = _RB * _D


def _sinusoid_table():
    position = np.arange(0, _MAX_LEN, dtype=np.float64)[:, None]
    div_term = np.exp(np.arange(0, _D, 2, dtype=np.float64)
                      * (-math.log(10000.0) / _D))
    enc = np.zeros((_MAX_LEN, _D), dtype=np.float32)
    enc[:, 0::2] = np.sin(position * div_term)
    enc[:, 1::2] = np.cos(position * div_term)
    return jnp.asarray(enc.reshape(-1))


def _sc_body(x_hbm, emb_hbm, enc_hbm, out_hbm,
             emb16_v, enc_v, x_v, o_v,
             ex0, ex1, sx0, sx1, so0, so1):
    c = lax.axis_index("c")
    s = lax.axis_index("s")
    wid = s * 2 + c
    base = wid * _RPW * _D
    enc_sems = (ex0, ex1)
    x_sems = (sx0, sx1)
    o_sems = (so0, so1)

    for r in range(_RB):
        pltpu.sync_copy(emb_hbm.at[0], emb16_v.at[pl.ds(r * _D, _D)])

    segs = [(k, b) for k in range(_NK) for b in range(_B)]
    x_descs = [None] * len(segs)
    o_descs = [None] * len(segs)
    enc_descs = [None] * _NK

    def start_x(seg):
        k, b = segs[seg]
        slot = seg % 2
        x_descs[seg] = pltpu.async_copy(
            x_hbm.at[b, pl.ds(base + k * _BLK, _BLK)], x_v.at[slot],
            x_sems[slot])

    def start_enc(k):
        enc_descs[k] = pltpu.async_copy(
            enc_hbm.at[pl.ds(base + k * _BLK, _BLK)], enc_v.at[k % 2],
            enc_sems[k % 2])

    start_enc(0)
    if _NK > 1:
        start_enc(1)
    start_x(0)
    start_x(1)

    for seg, (k, b) in enumerate(segs):
        slot = seg % 2
        kslot = k % 2
        if b == 0:
            enc_descs[k].wait()
            if 1 <= k < _NK - 1:
                start_enc(k + 1)
        x_descs[seg].wait()
        if seg >= 2:
            o_descs[seg - 2].wait()

        @plsc.parallel_loop(0, _BLK, step=16, unroll=8)
        def _(i, slot=slot, kslot=kslot):
            o_v[slot, pl.ds(i, 16)] = (x_v[slot, pl.ds(i, 16)]
                                       + enc_v[kslot, pl.ds(i, 16)]
                                       + emb16_v[pl.ds(i, 16)])

        o_descs[seg] = pltpu.async_copy(
            o_v.at[slot], out_hbm.at[b, pl.ds(base + k * _BLK, _BLK)],
            o_sems[slot])

        if seg + 2 < len(segs):
            start_x(seg + 2)

    o_descs[-2].wait()
    o_descs[-1].wait()


def kernel(x, node_emb):
    enc = _sinusoid_table()
    x2 = x.reshape(_B, _S * _D)
    out = pl.kernel(
        _sc_body,
        out_type=jax.ShapeDtypeStruct((_B, _S * _D), jnp.float32),
        mesh=plsc.VectorSubcoreMesh(core_axis_name="c", subcore_axis_name="s"),
        scratch_types=[
            pltpu.VMEM((_BLK,), jnp.float32),
            pltpu.VMEM((2, _BLK), jnp.float32),
            pltpu.VMEM((2, _BLK), jnp.float32),
            pltpu.VMEM((2, _BLK), jnp.float32),
            pltpu.SemaphoreType.DMA,
            pltpu.SemaphoreType.DMA,
            pltpu.SemaphoreType.DMA,
            pltpu.SemaphoreType.DMA,
            pltpu.SemaphoreType.DMA,
            pltpu.SemaphoreType.DMA,
        ],
    )(x2, node_emb, enc)
    return out.reshape(_B, _S, _D)


# speedup vs baseline: 6.4563x; 6.0138x over previous
import math

import jax
import jax.numpy as jnp
from jax.experimental import pallas as pl
from jax.experimental.pallas import tpu as pltpu

_B = 4
_S = 4096
_D = 1024
_MAX_LEN = 4096
_TS = 2048
_NS = _S // _TS
_LOG_FACTOR = -math.log(10000.0) / _D


def _dim_rows():
    d = jax.lax.broadcasted_iota(jnp.int32, (1, _D), 1)
    freq = jnp.exp(((d // 2) * 2).astype(jnp.float32) * _LOG_FACTOR)
    phase = (d % 2).astype(jnp.float32) * (math.pi / 2)
    return freq, phase


def _pe_kernel(x_ref, emb_ref, o_ref, v_ref, w_ref):
    s = pl.program_id(0)
    b = pl.program_id(1)

    @pl.when((s == 0) & (b == 0))
    def _init_pe():
        pos = jax.lax.broadcasted_iota(jnp.int32, (8, 1), 0).astype(jnp.float32)
        freq, phase = _dim_rows()
        e = emb_ref[0, :][None, :]
        angle = pos * freq + phase
        v_ref[0:8, :] = jnp.sin(angle) + e
        w_ref[0:8, :] = jnp.cos(angle)
        k = 8
        while k < _TS:
            c = jnp.cos(k * freq)
            sn = jnp.sin(k * freq)
            r1 = e * (1.0 - c)
            r2 = e * sn
            v0 = v_ref[0:k, :]
            w0 = w_ref[0:k, :]
            v_ref[k:2 * k, :] = v0 * c + w0 * sn + r1
            w_ref[k:2 * k, :] = w0 * c - v0 * sn + r2
            k *= 2

    @pl.when((s > 0) & (b == 0))
    def _advance_pe():
        freq, _ = _dim_rows()
        e = emb_ref[0, :][None, :]
        c = jnp.cos(_TS * freq)
        sn = jnp.sin(_TS * freq)
        r1 = e * (1.0 - c)
        r2 = e * sn
        v = v_ref[...]
        w = w_ref[...]
        v_ref[...] = v * c + w * sn + r1
        w_ref[...] = w * c - v * sn + r2

    o_ref[...] = x_ref[...] + v_ref[...][None, :, :]


def kernel(x, node_emb):
    return pl.pallas_call(
        _pe_kernel,
        grid=(_NS, _B),
        in_specs=[
            pl.BlockSpec((1, _TS, _D), lambda s, b: (b, s, 0)),
            pl.BlockSpec((5, _D), lambda s, b: (0, 0)),
        ],
        out_specs=pl.BlockSpec((1, _TS, _D), lambda s, b: (b, s, 0)),
        out_shape=jax.ShapeDtypeStruct((_B, _S, _D), jnp.float32),
        scratch_shapes=[
            pltpu.VMEM((_TS, _D), jnp.float32),
            pltpu.VMEM((_TS, _D), jnp.float32),
        ],
    )(x, node_emb)
